# baseline (speedup 1.0000x reference)
import functools

import jax
import jax.numpy as jnp
from jax import lax
from jax.experimental import pallas as pl
from jax.experimental.pallas import tpu as pltpu
from jax.experimental.pallas import tpu_sc as plsc

SEQ = 16384
DIM = 64
HALF = 32
MAX_LEN = 512
NC = 2
NS = 16
L = 16
CHUNK = SEQ // NS
ROWS = SEQ // (NC * NS)
INT_MAX = 2147483647

_mesh = plsc.VectorSubcoreMesh(core_axis_name="c", subcore_axis_name="s")


@functools.partial(
    pl.kernel,
    out_type=jax.ShapeDtypeStruct((SEQ // 2, 2 * DIM), jnp.float32),
    mesh=_mesh,
    compiler_params=pltpu.CompilerParams(needs_layout_passes=False,
                                         use_tc_tiling_on_sc=True),
    scratch_types=[
        pltpu.VMEM((CHUNK,), jnp.int32),
        pltpu.VMEM((CHUNK,), jnp.int32),
        pltpu.VMEM((L,), jnp.int32),
        pltpu.VMEM((L,), jnp.int32),
        pltpu.VMEM_SHARED((NS * L,), jnp.int32),
        pltpu.VMEM_SHARED((NS * L,), jnp.int32),
        pltpu.VMEM((NS * L,), jnp.int32),
        pltpu.VMEM((NS * L,), jnp.int32),
        pltpu.VMEM((2 * L,), jnp.int32),
        pltpu.VMEM((2 * L,), jnp.int32),
        pltpu.VMEM((MAX_LEN * HALF,), jnp.float32),
        pltpu.VMEM((MAX_LEN * HALF,), jnp.float32),
        pltpu.VMEM((ROWS // 2, 2 * DIM), jnp.float32),
        pltpu.SemaphoreType.DMA,
        pltpu.SemaphoreType.DMA,
        pltpu.SemaphoreType.DMA,
    ],
)
def _pe_kernel(cx, cy, x2d, wx, wy, out, cx_v, cy_v, stx, sty,
               minx_sh, miny_sh, mgx, mgy, redx, redy,
               wx_v, wy_v, xo_v, sem, sem_x0, sem_x1):
    c = lax.axis_index("c")
    s = lax.axis_index("s")
    row0 = s * CHUNK + c * ROWS
    half0 = pl.multiple_of(row0 // 2, 256)
    cp_x = pltpu.async_copy(x2d.at[pl.ds(half0, ROWS // 2)], xo_v, sem_x0)
    cp_wx = pltpu.async_copy(wx, wx_v, sem)
    cp_wy = pltpu.async_copy(wy, wy_v, sem)

    pltpu.sync_copy(cx.at[pl.ds(s * CHUNK, CHUNK)], cx_v)
    pltpu.sync_copy(cy.at[pl.ds(s * CHUNK, CHUNK)], cy_v)

    def min_body(j, carry):
        mx, my = carry
        vx = cx_v[pl.ds(j * L, L)]
        vy = cy_v[pl.ds(j * L, L)]
        return jnp.minimum(mx, vx), jnp.minimum(my, vy)

    init = (jnp.full((L,), INT_MAX, jnp.int32), jnp.full((L,), INT_MAX, jnp.int32))
    mx, my = lax.fori_loop(0, CHUNK // L, min_body, init)

    stx[...] = mx
    sty[...] = my
    pltpu.sync_copy(stx, minx_sh.at[pl.ds(s * L, L)])
    pltpu.sync_copy(sty, miny_sh.at[pl.ds(s * L, L)])
    plsc.subcore_barrier()
    pltpu.sync_copy(minx_sh, mgx)
    pltpu.sync_copy(miny_sh, mgy)
    vx = mgx[pl.ds(0, L)]
    vy = mgy[pl.ds(0, L)]
    for j in range(1, NS):
        vx = jnp.minimum(vx, mgx[pl.ds(j * L, L)])
        vy = jnp.minimum(vy, mgy[pl.ds(j * L, L)])
    redx[pl.ds(0, L)] = vx
    redx[pl.ds(L, L)] = vx
    redy[pl.ds(0, L)] = vy
    redy[pl.ds(L, L)] = vy
    gmx = redx[pl.ds(0, L)]
    gmy = redy[pl.ds(0, L)]
    for k in range(1, L):
        gmx = jnp.minimum(gmx, redx[pl.ds(k, L)])
        gmy = jnp.minimum(gmy, redy[pl.ds(k, L)])

    off = c * ROWS
    shift4 = jnp.full((L,), 4, jnp.int32)
    shift5 = jnp.full((L,), 5, jnp.int32)

    def add_group(j):
        vx = cx_v[pl.ds(off + j * L, L)]
        vy = cy_v[pl.ds(off + j * L, L)]
        gxv = lax.shift_left(lax.shift_right_logical(vx - gmx, shift4), shift5)
        gyv = lax.shift_left(lax.shift_right_logical(vy - gmy, shift4), shift5)
        for t in range(L):
            ox = gxv[t]
            oy = gyv[t]
            rr = (j * L + t) // 2
            h = (t % 2) * DIM
            xo_v[rr, pl.ds(h, L)] = (
                xo_v[rr, pl.ds(h, L)] + wx_v[pl.ds(ox, L)])
            xo_v[rr, pl.ds(h + 16, L)] = (
                xo_v[rr, pl.ds(h + 16, L)] + wx_v[pl.ds(ox + 16, L)])
            xo_v[rr, pl.ds(h + 32, L)] = (
                xo_v[rr, pl.ds(h + 32, L)] + wy_v[pl.ds(oy, L)])
            xo_v[rr, pl.ds(h + 48, L)] = (
                xo_v[rr, pl.ds(h + 48, L)] + wy_v[pl.ds(oy + 16, L)])

    cp_wx.wait()
    cp_wy.wait()
    cp_x.wait()

    @plsc.parallel_loop(0, ROWS // L, 1, unroll=1)
    def add_body(j):
        add_group(j)

    pltpu.sync_copy(xo_v, out.at[pl.ds(half0, ROWS // 2)])


def kernel(x, coords, Wx, Wy):
    out = _pe_kernel(coords[:, 1], coords[:, 2], x.reshape(SEQ // 2, 2 * DIM),
                     Wx.reshape(-1), Wy.reshape(-1))
    return out.reshape(SEQ, DIM)


# speedup vs baseline: 1.2066x; 1.2066x over previous
import functools

import jax
import jax.numpy as jnp
from jax import lax
from jax.experimental import pallas as pl
from jax.experimental.pallas import tpu as pltpu
from jax.experimental.pallas import tpu_sc as plsc

SEQ = 16384
DIM = 64
HALF = 32
MAX_LEN = 512
NC = 2
NS = 16
L = 16
CHUNK = SEQ // NS
ROWS = SEQ // (NC * NS)
INT_MAX = 2147483647

_mesh = plsc.VectorSubcoreMesh(core_axis_name="c", subcore_axis_name="s")


@functools.partial(
    pl.kernel,
    out_type=jax.ShapeDtypeStruct((SEQ, DIM), jnp.float32),
    mesh=_mesh,
    compiler_params=pltpu.CompilerParams(needs_layout_passes=False,
                                         use_tc_tiling_on_sc=True),
    scratch_types=[
        pltpu.VMEM((CHUNK,), jnp.int32),
        pltpu.VMEM((CHUNK,), jnp.int32),
        pltpu.VMEM((L,), jnp.int32),
        pltpu.VMEM((L,), jnp.int32),
        pltpu.VMEM_SHARED((NS * L,), jnp.int32),
        pltpu.VMEM_SHARED((NS * L,), jnp.int32),
        pltpu.VMEM((NS * L,), jnp.int32),
        pltpu.VMEM((NS * L,), jnp.int32),
        pltpu.VMEM((2 * L,), jnp.int32),
        pltpu.VMEM((2 * L,), jnp.int32),
        pltpu.VMEM((MAX_LEN * HALF,), jnp.float32),
        pltpu.VMEM((MAX_LEN * HALF,), jnp.float32),
        pltpu.VMEM((ROWS, DIM), jnp.float32),
        pltpu.SemaphoreType.DMA,
        pltpu.SemaphoreType.DMA,
        pltpu.SemaphoreType.DMA,
    ],
)
def _pe_kernel(cx, cy, x2d, wx, wy, out, cx_v, cy_v, stx, sty,
               minx_sh, miny_sh, mgx, mgy, redx, redy,
               wx_v, wy_v, xo_v, sem, sem_x0, sem_x1):
    c = lax.axis_index("c")
    s = lax.axis_index("s")
    row0 = pl.multiple_of(s * CHUNK + c * ROWS, ROWS)
    cp_x = pltpu.async_copy(x2d.at[pl.ds(row0, ROWS)], xo_v, sem_x0)
    cp_wx = pltpu.async_copy(wx, wx_v, sem)
    cp_wy = pltpu.async_copy(wy, wy_v, sem)

    pltpu.sync_copy(cx.at[pl.ds(s * CHUNK, CHUNK)], cx_v)
    pltpu.sync_copy(cy.at[pl.ds(s * CHUNK, CHUNK)], cy_v)

    def min_body(j, carry):
        mx, my = carry
        vx = cx_v[pl.ds(j * L, L)]
        vy = cy_v[pl.ds(j * L, L)]
        return jnp.minimum(mx, vx), jnp.minimum(my, vy)

    init = (jnp.full((L,), INT_MAX, jnp.int32), jnp.full((L,), INT_MAX, jnp.int32))
    mx, my = lax.fori_loop(0, CHUNK // L, min_body, init)

    stx[...] = mx
    sty[...] = my
    pltpu.sync_copy(stx, minx_sh.at[pl.ds(s * L, L)])
    pltpu.sync_copy(sty, miny_sh.at[pl.ds(s * L, L)])
    plsc.subcore_barrier()
    pltpu.sync_copy(minx_sh, mgx)
    pltpu.sync_copy(miny_sh, mgy)
    vx = mgx[pl.ds(0, L)]
    vy = mgy[pl.ds(0, L)]
    for j in range(1, NS):
        vx = jnp.minimum(vx, mgx[pl.ds(j * L, L)])
        vy = jnp.minimum(vy, mgy[pl.ds(j * L, L)])
    redx[pl.ds(0, L)] = vx
    redx[pl.ds(L, L)] = vx
    redy[pl.ds(0, L)] = vy
    redy[pl.ds(L, L)] = vy
    gmx = redx[pl.ds(0, L)]
    gmy = redy[pl.ds(0, L)]
    for k in range(1, L):
        gmx = jnp.minimum(gmx, redx[pl.ds(k, L)])
        gmy = jnp.minimum(gmy, redy[pl.ds(k, L)])

    off = c * ROWS
    shift4 = jnp.full((L,), 4, jnp.int32)
    shift5 = jnp.full((L,), 5, jnp.int32)

    def add_group(j):
        vx = cx_v[pl.ds(off + j * L, L)]
        vy = cy_v[pl.ds(off + j * L, L)]
        gxv = lax.shift_left(lax.shift_right_logical(vx - gmx, shift4), shift5)
        gyv = lax.shift_left(lax.shift_right_logical(vy - gmy, shift4), shift5)
        for t in range(L):
            ox = gxv[t]
            oy = gyv[t]
            r = j * L + t
            xo_v[r, pl.ds(0, L)] = xo_v[r, pl.ds(0, L)] + wx_v[pl.ds(ox, L)]
            xo_v[r, pl.ds(16, L)] = (
                xo_v[r, pl.ds(16, L)] + wx_v[pl.ds(ox + 16, L)])
            xo_v[r, pl.ds(32, L)] = xo_v[r, pl.ds(32, L)] + wy_v[pl.ds(oy, L)]
            xo_v[r, pl.ds(48, L)] = (
                xo_v[r, pl.ds(48, L)] + wy_v[pl.ds(oy + 16, L)])

    cp_wx.wait()
    cp_wy.wait()
    cp_x.wait()

    @plsc.parallel_loop(0, ROWS // L, 1, unroll=1)
    def add_body(j):
        add_group(j)

    pltpu.sync_copy(xo_v, out.at[pl.ds(row0, ROWS)])


def kernel(x, coords, Wx, Wy):
    return _pe_kernel(coords[:, 1], coords[:, 2], x,
                      Wx.reshape(-1), Wy.reshape(-1))


# speedup vs baseline: 1.2067x; 1.0001x over previous
import functools

import jax
import jax.numpy as jnp
from jax import lax
from jax.experimental import pallas as pl
from jax.experimental.pallas import tpu as pltpu
from jax.experimental.pallas import tpu_sc as plsc

SEQ = 16384
DIM = 64
HALF = 32
MAX_LEN = 512
NC = 2
NS = 16
L = 16
CHUNK = SEQ // NS
ROWS = SEQ // (NC * NS)
INT_MAX = 2147483647

_mesh = plsc.VectorSubcoreMesh(core_axis_name="c", subcore_axis_name="s")


@functools.partial(
    pl.kernel,
    out_type=jax.ShapeDtypeStruct((SEQ, DIM), jnp.float32),
    mesh=_mesh,
    compiler_params=pltpu.CompilerParams(needs_layout_passes=False,
                                         use_tc_tiling_on_sc=True),
    scratch_types=[
        pltpu.VMEM((CHUNK,), jnp.int32),
        pltpu.VMEM((CHUNK,), jnp.int32),
        pltpu.VMEM((L,), jnp.int32),
        pltpu.VMEM((L,), jnp.int32),
        pltpu.VMEM_SHARED((NS * L,), jnp.int32),
        pltpu.VMEM_SHARED((NS * L,), jnp.int32),
        pltpu.VMEM((NS * L,), jnp.int32),
        pltpu.VMEM((NS * L,), jnp.int32),
        pltpu.VMEM((2 * L,), jnp.int32),
        pltpu.VMEM((2 * L,), jnp.int32),
        pltpu.VMEM((MAX_LEN * HALF,), jnp.float32),
        pltpu.VMEM((MAX_LEN * HALF,), jnp.float32),
        pltpu.VMEM((ROWS, DIM), jnp.float32),
        pltpu.SemaphoreType.DMA,
        pltpu.SemaphoreType.DMA,
    ],
)
def _pe_kernel(cx, cy, x2d, wx, wy, out, cx_v, cy_v, stx, sty,
               minx_sh, miny_sh, mgx, mgy, redx, redy,
               wx_v, wy_v, xo_v, sem, sem_x0):
    c = lax.axis_index("c")
    s = lax.axis_index("s")
    row0 = pl.multiple_of(s * CHUNK + c * ROWS, ROWS)
    cp_x = pltpu.async_copy(x2d.at[pl.ds(row0, ROWS)], xo_v, sem_x0)
    cp_wx = pltpu.async_copy(wx, wx_v, sem)
    cp_wy = pltpu.async_copy(wy, wy_v, sem)

    pltpu.sync_copy(cx.at[pl.ds(s * CHUNK, CHUNK)], cx_v)
    pltpu.sync_copy(cy.at[pl.ds(s * CHUNK, CHUNK)], cy_v)

    def min_body(j, carry):
        mx, my = carry
        vx = cx_v[pl.ds(j * L, L)]
        vy = cy_v[pl.ds(j * L, L)]
        return jnp.minimum(mx, vx), jnp.minimum(my, vy)

    init = (jnp.full((L,), INT_MAX, jnp.int32), jnp.full((L,), INT_MAX, jnp.int32))
    mx, my = lax.fori_loop(0, CHUNK // L, min_body, init)

    stx[...] = mx
    sty[...] = my
    pltpu.sync_copy(stx, minx_sh.at[pl.ds(s * L, L)])
    pltpu.sync_copy(sty, miny_sh.at[pl.ds(s * L, L)])
    plsc.subcore_barrier()
    pltpu.sync_copy(minx_sh, mgx)
    pltpu.sync_copy(miny_sh, mgy)
    vx = mgx[pl.ds(0, L)]
    vy = mgy[pl.ds(0, L)]
    for j in range(1, NS):
        vx = jnp.minimum(vx, mgx[pl.ds(j * L, L)])
        vy = jnp.minimum(vy, mgy[pl.ds(j * L, L)])
    redx[pl.ds(0, L)] = vx
    redx[pl.ds(L, L)] = vx
    redy[pl.ds(0, L)] = vy
    redy[pl.ds(L, L)] = vy
    gmx = redx[pl.ds(0, L)]
    gmy = redy[pl.ds(0, L)]
    for k in range(1, L):
        gmx = jnp.minimum(gmx, redx[pl.ds(k, L)])
        gmy = jnp.minimum(gmy, redy[pl.ds(k, L)])

    off = c * ROWS
    shift4 = jnp.full((L,), 4, jnp.int32)
    shift5 = jnp.full((L,), 5, jnp.int32)

    def add_group(j):
        vx = cx_v[pl.ds(off + j * L, L)]
        vy = cy_v[pl.ds(off + j * L, L)]
        gxv = lax.shift_left(lax.shift_right_logical(vx - gmx, shift4), shift5)
        gyv = lax.shift_left(lax.shift_right_logical(vy - gmy, shift4), shift5)
        for t in range(L):
            ox = gxv[t]
            oy = gyv[t]
            r = j * L + t
            xo_v[r, pl.ds(0, L)] = xo_v[r, pl.ds(0, L)] + wx_v[pl.ds(ox, L)]
            xo_v[r, pl.ds(16, L)] = (
                xo_v[r, pl.ds(16, L)] + wx_v[pl.ds(ox + 16, L)])
            xo_v[r, pl.ds(32, L)] = xo_v[r, pl.ds(32, L)] + wy_v[pl.ds(oy, L)]
            xo_v[r, pl.ds(48, L)] = (
                xo_v[r, pl.ds(48, L)] + wy_v[pl.ds(oy + 16, L)])

    cp_wx.wait()
    cp_wy.wait()
    cp_x.wait()

    @plsc.parallel_loop(0, ROWS // L, 1, unroll=1)
    def add_body(j):
        add_group(j)

    pltpu.sync_copy(xo_v, out.at[pl.ds(row0, ROWS)])


def kernel(x, coords, Wx, Wy):
    return _pe_kernel(coords[:, 1], coords[:, 2], x,
                      Wx.reshape(-1), Wy.reshape(-1))
